# Initial kernel scaffold; baseline (speedup 1.0000x reference)
import functools

import jax
import jax.numpy as jnp
from jax import lax
from jax.experimental import pallas as pl
from jax.experimental.pallas import tpu as pltpu
from jax.experimental.pallas import tpu_sc as plsc

NUM_FIELDS = 26
VOCAB = 100000
EMBED_DIM = 32
NUM_FEATS = 13
BATCH = 16384
H1, H2 = 256, 128

NC, NS = 2, 16
NW = NC * NS
TOTAL_ROWS = BATCH * NUM_FIELDS
ROWS_PER_W = TOTAL_ROWS // NW
IDX_PER_DMA = 128
CHUNK = 1664
DMAS_PER_CHUNK = CHUNK // IDX_PER_DMA
CHUNKS_PER_W = ROWS_PER_W // CHUNK

BN_SCALE = float(1.0 / (1.0 + 1e-5) ** 0.5)


def _gather_body(table_hbm, idx_hbm, out_hbm, idx_v, rows_v, sem):
    wid = lax.axis_index("s") * NC + lax.axis_index("c")
    base = wid * ROWS_PER_W

    def chunk_step(k, _):
        row0 = base + k * CHUNK
        pltpu.sync_copy(idx_hbm.at[pl.ds(row0 // IDX_PER_DMA, DMAS_PER_CHUNK)],
                        idx_v)
        for j in range(DMAS_PER_CHUNK):
            pltpu.async_copy(table_hbm.at[idx_v.at[j]],
                             rows_v.at[pl.ds(j * IDX_PER_DMA, IDX_PER_DMA)],
                             sem)
        for j in range(DMAS_PER_CHUNK):
            pltpu.make_async_copy(table_hbm.at[idx_v.at[j]],
                                  rows_v.at[pl.ds(j * IDX_PER_DMA, IDX_PER_DMA)],
                                  sem).wait()
        pltpu.sync_copy(rows_v, out_hbm.at[pl.ds(row0, CHUNK)])
        return ()

    lax.fori_loop(0, CHUNKS_PER_W, chunk_step, ())


@jax.jit
def _sc_gather(table2d, idx2d):
    mesh = plsc.VectorSubcoreMesh(core_axis_name="c", subcore_axis_name="s",
                                  num_cores=NC, num_subcores=NS)
    return pl.kernel(
        _gather_body,
        out_type=jax.ShapeDtypeStruct((TOTAL_ROWS, EMBED_DIM), jnp.float32),
        mesh=mesh,
        scratch_types=[
            pltpu.VMEM((DMAS_PER_CHUNK, IDX_PER_DMA), jnp.int32),
            pltpu.VMEM((CHUNK, EMBED_DIM), jnp.float32),
            pltpu.SemaphoreType.DMA,
        ],
    )(table2d, idx2d)


BB = 2048


def _mlp_body(cat_ref, num_ref, w1a_ref, w1b_ref, b1_ref, w2_ref, b2_ref,
              w3_ref, b3_ref, out_ref):
    x_cat = cat_ref[...]
    x_num = num_ref[...] * BN_SCALE
    h = lax.dot_general(x_cat, w1a_ref[...], (((1,), (1,)), ((), ())),
                        preferred_element_type=jnp.float32)
    h = h + lax.dot_general(x_num, w1b_ref[...], (((1,), (1,)), ((), ())),
                            preferred_element_type=jnp.float32)
    h = jnp.maximum(h + b1_ref[...], 0.0)
    h = lax.dot_general(h, w2_ref[...], (((1,), (1,)), ((), ())),
                        preferred_element_type=jnp.float32)
    h = jnp.maximum(h + b2_ref[...], 0.0)
    o = lax.dot_general(h, w3_ref[...], (((1,), (1,)), ((), ())),
                        preferred_element_type=jnp.float32)
    out_ref[...] = o + b3_ref[...]


@jax.jit
def _tc_mlp(cat_vec, nums, w1a, w1b, b1, w2, b2, w3, b3):
    nblk = BATCH // BB
    full = lambda i: (0, 0)
    return pl.pallas_call(
        _mlp_body,
        grid=(nblk,),
        in_specs=[
            pl.BlockSpec((BB, NUM_FIELDS * EMBED_DIM), lambda i: (i, 0)),
            pl.BlockSpec((BB, NUM_FEATS), lambda i: (i, 0)),
            pl.BlockSpec((H1, NUM_FIELDS * EMBED_DIM), full),
            pl.BlockSpec((H1, NUM_FEATS), full),
            pl.BlockSpec((1, H1), full),
            pl.BlockSpec((H2, H1), full),
            pl.BlockSpec((1, H2), full),
            pl.BlockSpec((1, H2), full),
            pl.BlockSpec((1, 1), full),
        ],
        out_specs=pl.BlockSpec((BB, 1), lambda i: (i, 0)),
        out_shape=jax.ShapeDtypeStruct((BATCH, 1), jnp.float32),
    )(cat_vec, nums, w1a, w1b, b1, w2, b2, w3, b3)


def kernel(cats, nums, tables, W1, b1, W2, b2, W3, b3):
    cats = cats.astype(jnp.int32)
    flat_idx = cats + (jnp.arange(NUM_FIELDS, dtype=jnp.int32) * VOCAB)[None, :]
    idx2d = flat_idx.reshape(TOTAL_ROWS // IDX_PER_DMA, IDX_PER_DMA)
    table2d = tables.reshape(NUM_FIELDS * VOCAB, EMBED_DIM)

    rows = _sc_gather(table2d, idx2d)
    cat_vec = rows.reshape(BATCH, NUM_FIELDS * EMBED_DIM)

    w1a = W1[:, : NUM_FIELDS * EMBED_DIM]
    w1b = W1[:, NUM_FIELDS * EMBED_DIM:]
    out = _tc_mlp(cat_vec, nums, w1a, w1b, b1.reshape(1, H1),
                  W2, b2.reshape(1, H2), W3, b3.reshape(1, 1))
    return out.reshape(BATCH)


# speedup vs baseline: 2.0193x; 2.0193x over previous
import functools

import jax
import jax.numpy as jnp
from jax import lax
from jax.experimental import pallas as pl
from jax.experimental.pallas import tpu as pltpu
from jax.experimental.pallas import tpu_sc as plsc

NUM_FIELDS = 26
VOCAB = 100000
EMBED_DIM = 32
NUM_FEATS = 13
BATCH = 16384
H1, H2 = 256, 128

NC, NS = 2, 16
NW = NC * NS
TOTAL_ROWS = BATCH * NUM_FIELDS
ROWS_PER_W = TOTAL_ROWS // NW
IDX_PER_DMA = 128
CHUNK = 1024
DMAS_PER_CHUNK = CHUNK // IDX_PER_DMA
CHUNKS_PER_W = ROWS_PER_W // CHUNK

BN_SCALE = float(1.0 / (1.0 + 1e-5) ** 0.5)


def _gather_body(table_hbm, idx_hbm, out_hbm, idx_v, rows_v, sem):
    wid = lax.axis_index("s") * NC + lax.axis_index("c")
    base = wid * ROWS_PER_W

    def chunk_step(k, _):
        row0 = pl.multiple_of(base + k * CHUNK, CHUNK)
        pltpu.sync_copy(
            idx_hbm.at[pl.ds(pl.multiple_of(row0 // IDX_PER_DMA, DMAS_PER_CHUNK),
                             DMAS_PER_CHUNK)],
            idx_v)
        for j in range(DMAS_PER_CHUNK):
            pltpu.async_copy(table_hbm.at[idx_v.at[j]],
                             rows_v.at[pl.ds(j * IDX_PER_DMA, IDX_PER_DMA)],
                             sem)
        for j in range(DMAS_PER_CHUNK):
            pltpu.make_async_copy(table_hbm.at[idx_v.at[j]],
                                  rows_v.at[pl.ds(j * IDX_PER_DMA, IDX_PER_DMA)],
                                  sem).wait()
        pltpu.sync_copy(rows_v, out_hbm.at[pl.ds(row0, CHUNK)])
        return ()

    lax.fori_loop(0, CHUNKS_PER_W, chunk_step, ())


@jax.jit
def _sc_gather(table2d, idx2d):
    mesh = plsc.VectorSubcoreMesh(core_axis_name="c", subcore_axis_name="s",
                                  num_cores=NC, num_subcores=NS)
    return pl.kernel(
        _gather_body,
        out_type=jax.ShapeDtypeStruct((TOTAL_ROWS, EMBED_DIM), jnp.float32),
        mesh=mesh,
        scratch_types=[
            pltpu.VMEM((DMAS_PER_CHUNK, IDX_PER_DMA), jnp.int32),
            pltpu.VMEM((CHUNK, EMBED_DIM), jnp.float32),
            pltpu.SemaphoreType.DMA,
        ],
        compiler_params=pltpu.CompilerParams(use_tc_tiling_on_sc=False),
    )(table2d, idx2d)


BB = 2048


def _mlp_body(cat_ref, num_ref, w1a_ref, w1b_ref, b1_ref, w2_ref, b2_ref,
              w3_ref, b3_ref, out_ref):
    x_cat = cat_ref[...]
    x_num = num_ref[...] * BN_SCALE
    h = lax.dot_general(x_cat, w1a_ref[...], (((1,), (1,)), ((), ())),
                        preferred_element_type=jnp.float32)
    h = h + lax.dot_general(x_num, w1b_ref[...], (((1,), (1,)), ((), ())),
                            preferred_element_type=jnp.float32)
    h = jnp.maximum(h + b1_ref[...], 0.0)
    h = lax.dot_general(h, w2_ref[...], (((1,), (1,)), ((), ())),
                        preferred_element_type=jnp.float32)
    h = jnp.maximum(h + b2_ref[...], 0.0)
    o = lax.dot_general(h, w3_ref[...], (((1,), (0,)), ((), ())),
                        preferred_element_type=jnp.float32)
    out_ref[...] = o + b3_ref[0, 0]


@jax.jit
def _tc_mlp(cat_vec, nums, w1a, w1b, b1, w2, b2, w3, b3):
    nblk = BATCH // BB
    full = lambda i: (0, 0)
    return pl.pallas_call(
        _mlp_body,
        grid=(nblk,),
        in_specs=[
            pl.BlockSpec((BB, NUM_FIELDS * EMBED_DIM), lambda i: (i, 0)),
            pl.BlockSpec((BB, NUM_FEATS), lambda i: (i, 0)),
            pl.BlockSpec((H1, NUM_FIELDS * EMBED_DIM), full),
            pl.BlockSpec((H1, NUM_FEATS), full),
            pl.BlockSpec((1, H1), full),
            pl.BlockSpec((H2, H1), full),
            pl.BlockSpec((1, H2), full),
            pl.BlockSpec((H2, 1), full),
            pl.BlockSpec(memory_space=pltpu.SMEM),
        ],
        out_specs=pl.BlockSpec((BB, 1), lambda i: (i, 0)),
        out_shape=jax.ShapeDtypeStruct((BATCH, 1), jnp.float32),
    )(cat_vec, nums, w1a, w1b, b1, w2, b2, w3, b3)


def kernel(cats, nums, tables, W1, b1, W2, b2, W3, b3):
    cats = cats.astype(jnp.int32)
    flat_idx = cats + (jnp.arange(NUM_FIELDS, dtype=jnp.int32) * VOCAB)[None, :]
    idx2d = flat_idx.reshape(TOTAL_ROWS // IDX_PER_DMA, IDX_PER_DMA)
    table2d = tables.reshape(NUM_FIELDS * VOCAB, EMBED_DIM)

    rows = _sc_gather(table2d, idx2d)
    cat_vec = rows.reshape(BATCH, NUM_FIELDS * EMBED_DIM)

    w1a = W1[:, : NUM_FIELDS * EMBED_DIM]
    w1b = W1[:, NUM_FIELDS * EMBED_DIM:]
    out = _tc_mlp(cat_vec, nums, w1a, w1b, b1.reshape(1, H1),
                  W2, b2.reshape(1, H2), W3.reshape(H2, 1), b3.reshape(1, 1))
    return out.reshape(BATCH)
